# baseline (speedup 1.0000x reference)
import functools

import jax
import jax.numpy as jnp
from jax import lax
from jax.experimental import pallas as pl
from jax.experimental.pallas import tpu as pltpu
from jax.experimental.pallas import tpu_sc as plsc

N = 100000
H = 128
M = 1024
K = 3

_NW = 32
_CH = 3200
_NPAD = _NW * _CH
_ACC = M * K
_SL = _ACC // 16
_TAIL = N - (_NW - 1) * _CH


_BN = 3200
_NB = _NPAD // _BN
_TCTAIL = N - (_NB - 1) * _BN


def _proj_body(v_ref, w_ref, o_ref):
    i = pl.program_id(0)
    for k in range(K):
        yk = lax.dot_general(
            w_ref[...], v_ref[k],
            (((1,), (1,)), ((), ())),
            preferred_element_type=jnp.float32,
        )

        @pl.when(i < _NB - 1)
        def _(yk=yk, k=k):
            o_ref[k:k + 1, :] = yk

        @pl.when(i == _NB - 1)
        def _(yk=yk, k=k):
            col = lax.broadcasted_iota(jnp.int32, (1, _BN), 1)
            o_ref[k:k + 1, :] = jnp.where(col < _TCTAIL, yk, 0.0)


def _tc_project(vt, w):
    return pl.pallas_call(
        _proj_body,
        grid=(_NB,),
        in_specs=[
            pl.BlockSpec((K, _BN, H), lambda i: (0, i, 0)),
            pl.BlockSpec((1, H), lambda i: (0, 0)),
        ],
        out_specs=pl.BlockSpec((K, _BN), lambda i: (0, i)),
        out_shape=jax.ShapeDtypeStruct((K, _NPAD), jnp.float32),
    )(vt, w)


def _segsum_body(vals_hbm, ids_hbm, out_hbm, vals_v, ids_v, acc_v, sum_v,
                 tmp_v, shared, dsem):
    c = lax.axis_index("c")
    s = lax.axis_index("s")
    wid = c * 16 + s
    base = wid * _CH

    vcopies = [
        pltpu.async_copy(vals_hbm.at[pl.ds(k * _NPAD + base, _CH)],
                         vals_v.at[pl.ds(k * _CH, _CH)], dsem)
        for k in range(K)
    ]

    @pl.when(wid < _NW - 1)
    def _():
        pltpu.async_copy(ids_hbm.at[pl.ds(base, _CH)], ids_v, dsem).wait()

    z16i = jnp.zeros((16,), jnp.int32)

    @pl.when(wid == _NW - 1)
    def _():
        pltpu.async_copy(ids_hbm.at[pl.ds((_NW - 1) * _CH, _TAIL)],
                         ids_v.at[pl.ds(0, _TAIL)], dsem).wait()

        def _zpad(j, _):
            ids_v[pl.ds(_TAIL + j * 16, 16)] = z16i
            return 0

        lax.fori_loop(0, (_CH - _TAIL) // 16, _zpad, 0)

    z16 = jnp.zeros((16,), jnp.float32)
    for cp in vcopies:
        cp.wait()

    def _zero(j, _):
        acc_v[pl.ds(j * 16, 16)] = z16
        return 0

    lax.fori_loop(0, _ACC // 16, _zero, 0)

    for k in range(K):
        def _accum(g, _, k=k):
            for u in range(4):
                o = g * 64 + u * 16
                idv = ids_v[pl.ds(o, 16)]
                vals = vals_v[pl.ds(k * _CH + o, 16)]
                plsc.addupdate_scatter(acc_v, [idv * 3 + k], vals)
            return 0

        lax.fori_loop(0, _CH // 64, _accum, 0)

    pltpu.sync_copy(acc_v, shared.at[pl.ds(s * _ACC, _ACC)])
    plsc.subcore_barrier()

    fetches = [
        pltpu.async_copy(shared.at[pl.ds(j * _ACC + s * _SL, _SL)],
                         tmp_v.at[pl.ds(j * _SL, _SL)], dsem)
        for j in range(16)
    ]
    for cp in fetches:
        cp.wait()

    def _add(t, _):
        def _add1(j, x):
            return x + tmp_v[pl.ds(j * _SL + t * 16, 16)]

        sum_v[pl.ds(t * 16, 16)] = lax.fori_loop(0, 16, _add1,
                                                 jnp.zeros((16,), jnp.float32))
        return 0

    lax.fori_loop(0, _SL // 16, _add, 0)

    pltpu.sync_copy(sum_v, out_hbm.at[pl.ds(c * _ACC + s * _SL, _SL)])


def _sc_segsum(vals_flat, ids):
    f = pl.kernel(
        _segsum_body,
        mesh=plsc.VectorSubcoreMesh(core_axis_name="c", subcore_axis_name="s"),
        out_type=jax.ShapeDtypeStruct((2 * _ACC,), jnp.float32),
        compiler_params=pltpu.CompilerParams(needs_layout_passes=False),
        scratch_types=[
            pltpu.VMEM((K * _CH,), jnp.float32),
            pltpu.VMEM((_CH,), jnp.int32),
            pltpu.VMEM((_ACC,), jnp.float32),
            pltpu.VMEM((_SL,), jnp.float32),
            pltpu.VMEM((16 * _SL,), jnp.float32),
            pltpu.VMEM_SHARED((16 * _ACC,), jnp.float32),
            pltpu.SemaphoreType.DMA,
        ],
    )
    return f(vals_flat, ids)


def kernel(v, batch, W):
    vt = jnp.transpose(v, (2, 0, 1))
    mu_t = _tc_project(vt, W)
    out2 = _sc_segsum(mu_t.reshape(-1), batch.astype(jnp.int32))
    out2 = out2.reshape(2, _ACC)
    return (out2[0] + out2[1]).reshape(M, K)


# speedup vs baseline: 1.1671x; 1.1671x over previous
import jax
import jax.numpy as jnp
from jax import lax
from jax.experimental import pallas as pl
from jax.experimental.pallas import tpu as pltpu
from jax.experimental.pallas import tpu_sc as plsc

N = 100000
H = 128
M = 1024
K = 3

_NW = 32
_CHH = 1600
_NPH = _NW * _CHH
_ACC = M * K
_SL = _ACC // 16


_BN = 6400
_NBH = _NPH // _BN
_TCTAIL = N - _NPH - (_NBH - 1) * _BN


def _make_proj_body(mask_tail):
    def _proj_body(v_ref, w_ref, o_ref):
        i = pl.program_id(0)
        for k in range(K):
            yk = lax.dot_general(
                w_ref[...], v_ref[k],
                (((1,), (1,)), ((), ())),
                preferred_element_type=jnp.float32,
            )

            if not mask_tail:
                o_ref[k:k + 1, :] = yk
            else:
                @pl.when(i < _NBH - 1)
                def _(yk=yk, k=k):
                    o_ref[k:k + 1, :] = yk

                @pl.when(i == _NBH - 1)
                def _(yk=yk, k=k):
                    col = lax.broadcasted_iota(jnp.int32, (1, _BN), 1)
                    o_ref[k:k + 1, :] = jnp.where(col < _TCTAIL, yk, 0.0)

    return _proj_body


def _tc_project_half(vt, w, boff, mask_tail):
    return pl.pallas_call(
        _make_proj_body(mask_tail),
        grid=(_NBH,),
        in_specs=[
            pl.BlockSpec((K, _BN, H), lambda i: (0, i + boff, 0)),
            pl.BlockSpec((1, H), lambda i: (0, 0)),
        ],
        out_specs=pl.BlockSpec((K, _BN), lambda i: (0, i)),
        out_shape=jax.ShapeDtypeStruct((K, _NPH), jnp.float32),
    )(vt, w)


def _make_segsum_body(ids_off, full_until, partial_len):
    def _segsum_body(vals_hbm, ids_hbm, out_hbm, vals_v, ids_v, acc_v, sum_v,
                     tmp_v, shared, dsem):
        c = lax.axis_index("c")
        s = lax.axis_index("s")
        wid = c * 16 + s
        base = wid * _CHH

        vcopies = [
            pltpu.async_copy(vals_hbm.at[pl.ds(k * _NPH + base, _CHH)],
                             vals_v.at[pl.ds(k * _CHH, _CHH)], dsem)
            for k in range(K)
        ]

        z16i = jnp.zeros((16,), jnp.int32)

        if full_until >= _NW:
            pltpu.async_copy(ids_hbm.at[pl.ds(ids_off + base, _CHH)],
                             ids_v, dsem).wait()
        else:
            @pl.when(wid < full_until)
            def _():
                pltpu.async_copy(ids_hbm.at[pl.ds(ids_off + base, _CHH)],
                                 ids_v, dsem).wait()

            @pl.when(wid == full_until)
            def _():
                pltpu.async_copy(
                    ids_hbm.at[pl.ds(ids_off + full_until * _CHH,
                                     partial_len)],
                    ids_v.at[pl.ds(0, partial_len)], dsem).wait()

                def _zpad(j, _):
                    ids_v[pl.ds(partial_len + j * 16, 16)] = z16i
                    return 0

                lax.fori_loop(0, (_CHH - partial_len) // 16, _zpad, 0)

            @pl.when(wid > full_until)
            def _():
                def _zall(j, _):
                    ids_v[pl.ds(j * 16, 16)] = z16i
                    return 0

                lax.fori_loop(0, _CHH // 16, _zall, 0)

        z16 = jnp.zeros((16,), jnp.float32)
        for cp in vcopies:
            cp.wait()

        def _zero(j, _):
            acc_v[pl.ds(j * 16, 16)] = z16
            return 0

        lax.fori_loop(0, _ACC // 16, _zero, 0)

        for k in range(K):
            def _accum(g, _, k=k):
                for u in range(4):
                    o = g * 64 + u * 16
                    idv = ids_v[pl.ds(o, 16)]
                    vals = vals_v[pl.ds(k * _CHH + o, 16)]
                    plsc.addupdate_scatter(acc_v, [idv * 3 + k], vals)
                return 0

            lax.fori_loop(0, _CHH // 64, _accum, 0)

        pltpu.sync_copy(acc_v, shared.at[pl.ds(s * _ACC, _ACC)])
        plsc.subcore_barrier()

        fetches = [
            pltpu.async_copy(shared.at[pl.ds(j * _ACC + s * _SL, _SL)],
                             tmp_v.at[pl.ds(j * _SL, _SL)], dsem)
            for j in range(16)
        ]
        for cp in fetches:
            cp.wait()

        def _add(t, _):
            def _add1(j, x):
                return x + tmp_v[pl.ds(j * _SL + t * 16, 16)]

            sum_v[pl.ds(t * 16, 16)] = lax.fori_loop(
                0, 16, _add1, jnp.zeros((16,), jnp.float32))
            return 0

        lax.fori_loop(0, _SL // 16, _add, 0)

        pltpu.sync_copy(sum_v, out_hbm.at[pl.ds(c * _ACC + s * _SL, _SL)])

    return _segsum_body


def _sc_segsum(vals_flat, ids, ids_off, full_until, partial_len):
    f = pl.kernel(
        _make_segsum_body(ids_off, full_until, partial_len),
        mesh=plsc.VectorSubcoreMesh(core_axis_name="c", subcore_axis_name="s"),
        out_type=jax.ShapeDtypeStruct((2 * _ACC,), jnp.float32),
        compiler_params=pltpu.CompilerParams(needs_layout_passes=False),
        scratch_types=[
            pltpu.VMEM((K * _CHH,), jnp.float32),
            pltpu.VMEM((_CHH,), jnp.int32),
            pltpu.VMEM((_ACC,), jnp.float32),
            pltpu.VMEM((_SL,), jnp.float32),
            pltpu.VMEM((16 * _SL,), jnp.float32),
            pltpu.VMEM_SHARED((16 * _ACC,), jnp.float32),
            pltpu.SemaphoreType.DMA,
        ],
    )
    return f(vals_flat, ids)


def kernel(v, batch, W):
    vt = jnp.transpose(v, (2, 0, 1))
    ids = batch.astype(jnp.int32)

    mu_a = _tc_project_half(vt, W, 0, False)
    out_a = _sc_segsum(mu_a.reshape(-1), ids, 0, _NW, 0)

    mu_b = _tc_project_half(vt, W, _NBH, True)
    out_b = _sc_segsum(mu_b.reshape(-1), ids, _NPH, 30, 800)

    tot = (out_a.reshape(2, _ACC) + out_b.reshape(2, _ACC)).sum(0)
    return tot.reshape(M, K)


# speedup vs baseline: 1.1812x; 1.0121x over previous
import jax
import jax.numpy as jnp
from jax import lax
from jax.experimental import pallas as pl
from jax.experimental.pallas import tpu as pltpu
from jax.experimental.pallas import tpu_sc as plsc

N = 100000
H = 128
M = 1024
K = 3

_NW = 32
_ACC = M * K
_SL = _ACC // 16


_BN = 6400
_NBA = 12
_NBB = 4
_NPA = _NBA * _BN
_NPB = _NBB * _BN
_CHA = _NPA // _NW
_CHB = _NPB // _NW
_TCTAIL = N - _NPA - (_NBB - 1) * _BN


def _make_proj_body(nb, mask_tail):
    def _proj_body(v_ref, w_ref, o_ref):
        i = pl.program_id(0)
        for k in range(K):
            yk = lax.dot_general(
                w_ref[...], v_ref[k],
                (((1,), (1,)), ((), ())),
                preferred_element_type=jnp.float32,
            )

            if not mask_tail:
                o_ref[k:k + 1, :] = yk
            else:
                @pl.when(i < nb - 1)
                def _(yk=yk, k=k):
                    o_ref[k:k + 1, :] = yk

                @pl.when(i == nb - 1)
                def _(yk=yk, k=k):
                    col = lax.broadcasted_iota(jnp.int32, (1, _BN), 1)
                    o_ref[k:k + 1, :] = jnp.where(col < _TCTAIL, yk, 0.0)

    return _proj_body


def _tc_project_part(vt, w, boff, nb, mask_tail):
    return pl.pallas_call(
        _make_proj_body(nb, mask_tail),
        grid=(nb,),
        in_specs=[
            pl.BlockSpec((K, _BN, H), lambda i: (0, i + boff, 0)),
            pl.BlockSpec((1, H), lambda i: (0, 0)),
        ],
        out_specs=pl.BlockSpec((K, _BN), lambda i: (0, i)),
        out_shape=jax.ShapeDtypeStruct((K, nb * _BN), jnp.float32),
    )(vt, w)


def _make_segsum_body(chh, nph, ids_off, full_until, partial_len, unroll):
    def _segsum_body(vals_hbm, ids_hbm, out_hbm, vals_v, ids_v, acc_v, sum_v,
                     tmp_v, shared, dsem):
        c = lax.axis_index("c")
        s = lax.axis_index("s")
        wid = c * 16 + s
        base = wid * chh

        vcopies = [
            pltpu.async_copy(vals_hbm.at[pl.ds(k * nph + base, chh)],
                             vals_v.at[pl.ds(k * chh, chh)], dsem)
            for k in range(K)
        ]

        z16i = jnp.zeros((16,), jnp.int32)

        if full_until >= _NW:
            pltpu.async_copy(ids_hbm.at[pl.ds(ids_off + base, chh)],
                             ids_v, dsem).wait()
        else:
            @pl.when(wid < full_until)
            def _():
                pltpu.async_copy(ids_hbm.at[pl.ds(ids_off + base, chh)],
                                 ids_v, dsem).wait()

            if partial_len:
                @pl.when(wid == full_until)
                def _():
                    pltpu.async_copy(
                        ids_hbm.at[pl.ds(ids_off + full_until * chh,
                                         partial_len)],
                        ids_v.at[pl.ds(0, partial_len)], dsem).wait()

                    def _zpad(j, _):
                        ids_v[pl.ds(partial_len + j * 16, 16)] = z16i
                        return 0

                    lax.fori_loop(0, (chh - partial_len) // 16, _zpad, 0)

            @pl.when(wid >= full_until + (1 if partial_len else 0))
            def _():
                def _zall(j, _):
                    ids_v[pl.ds(j * 16, 16)] = z16i
                    return 0

                lax.fori_loop(0, chh // 16, _zall, 0)

        z16 = jnp.zeros((16,), jnp.float32)
        for cp in vcopies:
            cp.wait()

        def _zero(j, _):
            acc_v[pl.ds(j * 16, 16)] = z16
            return 0

        lax.fori_loop(0, _ACC // 16, _zero, 0)

        for k in range(K):
            def _accum(g, _, k=k):
                for u in range(unroll):
                    o = g * (16 * unroll) + u * 16
                    idv = ids_v[pl.ds(o, 16)]
                    vals = vals_v[pl.ds(k * chh + o, 16)]
                    plsc.addupdate_scatter(acc_v, [idv * 3 + k], vals)
                return 0

            lax.fori_loop(0, chh // (16 * unroll), _accum, 0)

        pltpu.sync_copy(acc_v, shared.at[pl.ds(s * _ACC, _ACC)])
        plsc.subcore_barrier()

        fetches = [
            pltpu.async_copy(shared.at[pl.ds(j * _ACC + s * _SL, _SL)],
                             tmp_v.at[pl.ds(j * _SL, _SL)], dsem)
            for j in range(16)
        ]
        for cp in fetches:
            cp.wait()

        def _add(t, _):
            def _add1(j, x):
                return x + tmp_v[pl.ds(j * _SL + t * 16, 16)]

            sum_v[pl.ds(t * 16, 16)] = lax.fori_loop(
                0, 16, _add1, jnp.zeros((16,), jnp.float32))
            return 0

        lax.fori_loop(0, _SL // 16, _add, 0)

        pltpu.sync_copy(sum_v, out_hbm.at[pl.ds(c * _ACC + s * _SL, _SL)])

    return _segsum_body


def _sc_segsum(vals_flat, ids, chh, ids_off, full_until, partial_len):
    unroll = next(u for u in (4, 3, 2, 1) if chh % (16 * u) == 0)
    f = pl.kernel(
        _make_segsum_body(chh, chh * _NW, ids_off, full_until, partial_len,
                          unroll),
        mesh=plsc.VectorSubcoreMesh(core_axis_name="c", subcore_axis_name="s"),
        out_type=jax.ShapeDtypeStruct((2 * _ACC,), jnp.float32),
        compiler_params=pltpu.CompilerParams(needs_layout_passes=False),
        scratch_types=[
            pltpu.VMEM((K * chh,), jnp.float32),
            pltpu.VMEM((chh,), jnp.int32),
            pltpu.VMEM((_ACC,), jnp.float32),
            pltpu.VMEM((_SL,), jnp.float32),
            pltpu.VMEM((16 * _SL,), jnp.float32),
            pltpu.VMEM_SHARED((16 * _ACC,), jnp.float32),
            pltpu.SemaphoreType.DMA,
        ],
    )
    return f(vals_flat, ids)


def kernel(v, batch, W):
    vt = jnp.transpose(v, (2, 0, 1))
    ids = batch.astype(jnp.int32)

    mu_a = _tc_project_part(vt, W, 0, _NBA, False)
    out_a = _sc_segsum(mu_a.reshape(-1), ids, _CHA, 0, _NW, 0)

    mu_b = _tc_project_part(vt, W, _NBA, _NBB, True)
    out_b = _sc_segsum(mu_b.reshape(-1), ids, _CHB, _NPA, 29, 0)

    tot = (out_a.reshape(2, _ACC) + out_b.reshape(2, _ACC)).sum(0)
    return tot.reshape(M, K)
